# Initial kernel scaffold; baseline (speedup 1.0000x reference)
import jax
import jax.numpy as jnp
from jax.experimental import pallas as pl


def kernel(inputs, hx, adj, node_index, W_ru, b_ru, W_c, b_c):
    raise NotImplementedError("write your pallas kernel here")


# speedup vs baseline: 1.2205x; 1.2205x over previous
import functools

import jax
import jax.numpy as jnp
from jax.experimental import pallas as pl


def _pick_bm(n, target):
    for bm in (target, 400, 200, 80, 40, 16, 8):
        if bm <= target and n % bm == 0:
            return bm
    return n


def _p1_body(bm, a_ref, x0_ref, x1_ref, dinv_ref, ahat_ref):
    i = pl.program_id(0)
    a = a_ref[...]
    d = jnp.sum(a, axis=1, keepdims=True) + 1.0
    dinv = jnp.where(d > 0.0, 1.0 / d, 0.0)
    ahat = (a * dinv).astype(jnp.bfloat16)
    ahat_ref[...] = ahat
    dinv_ref[...] = dinv
    xb = x0_ref[...]
    contrib = jax.lax.dot_general(
        ahat, xb.astype(jnp.bfloat16), (((0,), (0,)), ((), ())),
        preferred_element_type=jnp.float32)

    @pl.when(i == 0)
    def _():
        x1_ref[...] = jnp.zeros_like(x1_ref)

    x1_ref[...] += contrib
    x1_ref[pl.ds(i * bm, bm), :] += dinv * xb


def _diff_body(bm, finalize, ahat_ref, xb_ref, dinvb_ref, xf_ref, out_ref):
    i = pl.program_id(0)
    xb = xb_ref[...]
    contrib = jax.lax.dot_general(
        ahat_ref[...], xb.astype(jnp.bfloat16), (((0,), (0,)), ((), ())),
        preferred_element_type=jnp.float32)

    @pl.when(i == 0)
    def _():
        out_ref[...] = jnp.zeros_like(out_ref)

    out_ref[...] += contrib
    out_ref[pl.ds(i * bm, bm), :] += dinvb_ref[...] * xb
    if finalize:
        @pl.when(i == pl.num_programs(0) - 1)
        def _():
            out_ref[...] = 2.0 * out_ref[...] - xf_ref[...]


def _gate_body(fh, x0_ref, x1_ref, x2_ref, hx_ref, w_ref, b_ref,
               rhx_ref, u_ref):
    xcat = jnp.concatenate([x0_ref[...], x1_ref[...], x2_ref[...]], axis=1)
    v = jnp.dot(xcat, w_ref[...], preferred_element_type=jnp.float32)
    v = jax.nn.sigmoid(v + b_ref[...])
    rhx_ref[...] = v[:, :fh] * hx_ref[...]
    u_ref[...] = v[:, fh:]


def _final_body(fin, x0_ref, x1_ref, x2_ref, rhx_ref, x1s_ref, x2s_ref,
                hx_ref, u_ref, w_ref, b_ref, out_ref):
    xcat = jnp.concatenate([
        x0_ref[:, :fin], rhx_ref[...],
        x1_ref[:, :fin], x1s_ref[...],
        x2_ref[:, :fin], x2s_ref[...]], axis=1)
    c = jnp.dot(xcat, w_ref[...], preferred_element_type=jnp.float32)
    c = jnp.tanh(c + b_ref[...])
    u = u_ref[...]
    out_ref[...] = u * hx_ref[...] + (1.0 - u) * c


def _first_pass(adj, x0, bm):
    n, f = x0.shape
    grid = (n // bm,)
    return pl.pallas_call(
        functools.partial(_p1_body, bm),
        grid=grid,
        in_specs=[
            pl.BlockSpec((bm, n), lambda i: (i, 0)),
            pl.BlockSpec((bm, f), lambda i: (i, 0)),
        ],
        out_specs=[
            pl.BlockSpec((n, f), lambda i: (0, 0)),
            pl.BlockSpec((bm, 1), lambda i: (i, 0)),
            pl.BlockSpec((bm, n), lambda i: (i, 0)),
        ],
        out_shape=[
            jax.ShapeDtypeStruct((n, f), jnp.float32),
            jax.ShapeDtypeStruct((n, 1), jnp.float32),
            jax.ShapeDtypeStruct((n, n), jnp.bfloat16),
        ],
    )(adj, x0)


def _diffusion_pass(ahat, x, dinv, x_prev, finalize, bm):
    n, f = x.shape
    grid = (n // bm,)
    return pl.pallas_call(
        functools.partial(_diff_body, bm, finalize),
        grid=grid,
        in_specs=[
            pl.BlockSpec((bm, n), lambda i: (i, 0)),
            pl.BlockSpec((bm, f), lambda i: (i, 0)),
            pl.BlockSpec((bm, 1), lambda i: (i, 0)),
            pl.BlockSpec((n, f), lambda i: (0, 0)),
        ],
        out_specs=pl.BlockSpec((n, f), lambda i: (0, 0)),
        out_shape=jax.ShapeDtypeStruct((n, f), jnp.float32),
    )(ahat, x, dinv, x_prev)


def kernel(inputs, hx, adj, node_index, W_ru, b_ru, W_c, b_c):
    n = adj.shape[0]
    f_in = inputs.shape[1] // n
    f_h = hx.shape[1] // n
    f0 = f_in + f_h
    bm1 = _pick_bm(n, 200)
    bm = _pick_bm(n, 400)

    inp_nm = inputs.reshape(n, f_in)
    hx_nm = hx.reshape(n, f_h)
    x0 = jnp.concatenate([inp_nm, hx_nm], axis=1)

    w_ru = W_ru.reshape(f0, 3, 2 * f_h).transpose(1, 0, 2).reshape(3 * f0, 2 * f_h)
    w_c = W_c.reshape(f0, 3, f_h).transpose(1, 0, 2).reshape(3 * f0, f_h)

    x1, dinv, ahat = _first_pass(adj, x0, bm1)
    x2 = _diffusion_pass(ahat, x1, dinv, x0, True, bm)

    rhx, u = pl.pallas_call(
        functools.partial(_gate_body, f_h),
        out_shape=[
            jax.ShapeDtypeStruct((n, f_h), jnp.float32),
            jax.ShapeDtypeStruct((n, f_h), jnp.float32),
        ],
    )(x0, x1, x2, hx_nm, w_ru, b_ru.reshape(1, -1))

    x1s = _diffusion_pass(ahat, rhx, dinv, rhx, False, bm)
    x2s = _diffusion_pass(ahat, x1s, dinv, rhx, True, bm)

    new = pl.pallas_call(
        functools.partial(_final_body, f_in),
        out_shape=jax.ShapeDtypeStruct((n, f_h), jnp.float32),
    )(x0, x1, x2, rhx, x1s, x2s, hx_nm, u, w_c, b_c.reshape(1, -1))

    return new.reshape(1, n * f_h)


# speedup vs baseline: 1.4513x; 1.1891x over previous
import functools

import jax
import jax.numpy as jnp
from jax.experimental import pallas as pl


def _pick_bm(n, target):
    for bm in (target, 400, 200, 80, 40, 16, 8):
        if bm <= target and n % bm == 0:
            return bm
    return n


def _p1_body(a_ref, x0_ref, s1_ref, dinv_ref, ahat_ref):
    i = pl.program_id(0)
    a = a_ref[...]
    d = jnp.sum(a, axis=1, keepdims=True) + 1.0
    dinv = jnp.where(d > 0.0, 1.0 / d, 0.0)
    ahat = (a * dinv).astype(jnp.bfloat16)
    ahat_ref[...] = ahat
    dinv_ref[...] = dinv
    contrib = jax.lax.dot_general(
        x0_ref[...].astype(jnp.bfloat16), ahat, (((0,), (0,)), ((), ())),
        preferred_element_type=jnp.float32)

    @pl.when(i == 0)
    def _():
        s1_ref[...] = jnp.zeros_like(s1_ref)

    s1_ref[...] += contrib


def _diff_body(finalize, ahat_ref, xb_ref, xf_ref, xprev_ref, dinv_ref,
               out_ref):
    i = pl.program_id(0)
    contrib = jax.lax.dot_general(
        xb_ref[...].astype(jnp.bfloat16), ahat_ref[...],
        (((0,), (0,)), ((), ())), preferred_element_type=jnp.float32)

    @pl.when(i == 0)
    def _():
        out_ref[...] = jnp.zeros_like(out_ref)

    out_ref[...] += contrib

    @pl.when(i == pl.num_programs(0) - 1)
    def _():
        s = out_ref[...] + dinv_ref[...] * xf_ref[...]
        if finalize:
            s = 2.0 * s - xprev_ref[...]
        out_ref[...] = s


def _fixup_body(s_ref, dinv_ref, xf_ref, out_ref):
    out_ref[...] = s_ref[...] + dinv_ref[...] * xf_ref[...]


def _gate_body(fh, x0_ref, x1_ref, x2_ref, hx_ref, w_ref, b_ref,
               rhx_ref, u_ref):
    xcat = jnp.concatenate([x0_ref[...], x1_ref[...], x2_ref[...]], axis=0)
    v = jnp.dot(w_ref[...], xcat, preferred_element_type=jnp.float32)
    v = jax.nn.sigmoid(v + b_ref[...])
    rhx_ref[...] = v[:fh, :] * hx_ref[...]
    u_ref[...] = v[fh:, :]


def _final_body(fin, x0_ref, x1_ref, x2_ref, rhx_ref, x1s_ref, x2s_ref,
                hx_ref, u_ref, w_ref, b_ref, out_ref):
    xcat = jnp.concatenate([
        x0_ref[:fin, :], rhx_ref[...],
        x1_ref[:fin, :], x1s_ref[...],
        x2_ref[:fin, :], x2s_ref[...]], axis=0)
    c = jnp.dot(w_ref[...], xcat, preferred_element_type=jnp.float32)
    c = jnp.tanh(c + b_ref[...])
    u = u_ref[...]
    out_ref[...] = u * hx_ref[...] + (1.0 - u) * c


def _first_pass(adj, x0_nm, bm):
    n, f = x0_nm.shape
    return pl.pallas_call(
        _p1_body,
        grid=(n // bm,),
        in_specs=[
            pl.BlockSpec((bm, n), lambda i: (i, 0)),
            pl.BlockSpec((bm, f), lambda i: (i, 0)),
        ],
        out_specs=[
            pl.BlockSpec((f, n), lambda i: (0, 0)),
            pl.BlockSpec((bm, 1), lambda i: (i, 0)),
            pl.BlockSpec((bm, n), lambda i: (i, 0)),
        ],
        out_shape=[
            jax.ShapeDtypeStruct((f, n), jnp.float32),
            jax.ShapeDtypeStruct((n, 1), jnp.float32),
            jax.ShapeDtypeStruct((n, n), jnp.bfloat16),
        ],
    )(adj, x0_nm)


def _diffusion_pass(ahat, x_nm, xt, xprevt, dinv_row, finalize, bm):
    f, n = xt.shape
    return pl.pallas_call(
        functools.partial(_diff_body, finalize),
        grid=(n // bm,),
        in_specs=[
            pl.BlockSpec((bm, n), lambda i: (i, 0)),
            pl.BlockSpec((bm, f), lambda i: (i, 0)),
            pl.BlockSpec((f, n), lambda i: (0, 0)),
            pl.BlockSpec((f, n), lambda i: (0, 0)),
            pl.BlockSpec((1, n), lambda i: (0, 0)),
        ],
        out_specs=pl.BlockSpec((f, n), lambda i: (0, 0)),
        out_shape=jax.ShapeDtypeStruct((f, n), jnp.float32),
    )(ahat, x_nm, xt, xprevt, dinv_row)


def kernel(inputs, hx, adj, node_index, W_ru, b_ru, W_c, b_c):
    n = adj.shape[0]
    f_in = inputs.shape[1] // n
    f_h = hx.shape[1] // n
    f0 = f_in + f_h
    bm1 = _pick_bm(n, 200)
    bm = _pick_bm(n, 400)

    x0_nm = jnp.concatenate([inputs.reshape(n, f_in), hx.reshape(n, f_h)],
                            axis=1)
    x0t = x0_nm.T
    hx_t = hx.reshape(n, f_h).T

    w_ru = W_ru.reshape(f0, 3, 2 * f_h).transpose(1, 0, 2).reshape(3 * f0, 2 * f_h).T
    w_c = W_c.reshape(f0, 3, f_h).transpose(1, 0, 2).reshape(3 * f0, f_h).T

    s1t, dinv_col, ahat = _first_pass(adj, x0_nm, bm1)
    dinv_row = dinv_col.reshape(1, n)

    x1t = pl.pallas_call(
        _fixup_body,
        out_shape=jax.ShapeDtypeStruct((f0, n), jnp.float32),
    )(s1t, dinv_row, x0t)

    x2t = _diffusion_pass(ahat, x1t.T, x1t, x0t, dinv_row, True, bm)

    rhxt, ut = pl.pallas_call(
        functools.partial(_gate_body, f_h),
        out_shape=[
            jax.ShapeDtypeStruct((f_h, n), jnp.float32),
            jax.ShapeDtypeStruct((f_h, n), jnp.float32),
        ],
    )(x0t, x1t, x2t, hx_t, w_ru, b_ru.reshape(-1, 1))

    x1st = _diffusion_pass(ahat, rhxt.T, rhxt, rhxt, dinv_row, False, bm)

    x2st = _diffusion_pass(ahat, x1st.T, x1st, rhxt, dinv_row, True, bm)

    newt = pl.pallas_call(
        functools.partial(_final_body, f_in),
        out_shape=jax.ShapeDtypeStruct((f_h, n), jnp.float32),
    )(x0t, x1t, x2t, rhxt, x1st, x2st, hx_t, ut, w_c, b_c.reshape(-1, 1))

    return newt.T.reshape(1, n * f_h)


# speedup vs baseline: 1.5505x; 1.0684x over previous
import functools

import jax
import jax.numpy as jnp
from jax.experimental import pallas as pl
from jax.experimental.pallas import tpu as pltpu


def _pick_bm(n, target):
    for bm in (target, 400, 200, 80, 40, 16, 8):
        if bm <= target and n % bm == 0:
            return bm
    return n


def _p1_body(a_ref, x0_ref, s1_ref, dinv_ref, ahat_ref):
    i = pl.program_id(0)
    a = a_ref[...]
    d = jnp.sum(a, axis=1, keepdims=True) + 1.0
    dinv = jnp.where(d > 0.0, 1.0 / d, 0.0)
    ahat = (a * dinv).astype(jnp.bfloat16)
    ahat_ref[...] = ahat
    dinv_ref[...] = dinv
    contrib = jax.lax.dot_general(
        x0_ref[...].astype(jnp.bfloat16), ahat, (((0,), (0,)), ((), ())),
        preferred_element_type=jnp.float32)

    @pl.when(i == 0)
    def _():
        s1_ref[...] = jnp.zeros_like(s1_ref)

    s1_ref[...] += contrib


def _first_pass(adj, x0_nm, bm):
    n, f = x0_nm.shape
    return pl.pallas_call(
        _p1_body,
        grid=(n // bm,),
        in_specs=[
            pl.BlockSpec((bm, n), lambda i: (i, 0)),
            pl.BlockSpec((bm, f), lambda i: (i, 0)),
        ],
        out_specs=[
            pl.BlockSpec((f, n), lambda i: (0, 0)),
            pl.BlockSpec((bm, 1), lambda i: (i, 0)),
            pl.BlockSpec((bm, n), lambda i: (i, 0)),
        ],
        out_shape=[
            jax.ShapeDtypeStruct((f, n), jnp.float32),
            jax.ShapeDtypeStruct((n, 1), jnp.float32),
            jax.ShapeDtypeStruct((n, n), jnp.bfloat16),
        ],
    )(adj, x0_nm)


def _main_body(bm, fin, fh,
               ahat_ref, s1_ref, x0t_ref, dinv_ref, hxt_ref,
               wru_ref, bru_ref, wc_ref, bc_ref,
               out_ref,
               xnm_ref, x1t_ref, x2t_ref, x1st_ref, rhxt_ref, ut_ref,
               acc_ref):
    p = pl.program_id(0)
    i = pl.program_id(1)
    last = pl.num_programs(1) - 1
    dinv = dinv_ref[...]

    @pl.when((p == 0) & (i == 0))
    def _():
        x1t = s1_ref[...] + dinv * x0t_ref[...]
        x1t_ref[...] = x1t
        xnm_ref[...] = jnp.transpose(x1t.astype(jnp.bfloat16), (1, 0))

    ahat = ahat_ref[...]

    @pl.when(p == 0)
    def _():
        contrib = jax.lax.dot_general(
            xnm_ref[pl.ds(i * bm, bm), :], ahat, (((0,), (0,)), ((), ())),
            preferred_element_type=jnp.float32)

        @pl.when(i == 0)
        def _():
            acc_ref[...] = contrib

        @pl.when(i > 0)
        def _():
            acc_ref[...] += contrib

    @pl.when(p > 0)
    def _():
        contrib = jax.lax.dot_general(
            xnm_ref[pl.ds(i * bm, bm), :fh], ahat, (((0,), (0,)), ((), ())),
            preferred_element_type=jnp.float32)

        @pl.when(i == 0)
        def _():
            acc_ref[:fh, :] = contrib

        @pl.when(i > 0)
        def _():
            acc_ref[:fh, :] += contrib

    @pl.when((p == 0) & (i == last))
    def _():
        x1t = x1t_ref[...]
        x2t = 2.0 * (acc_ref[...] + dinv * x1t) - x0t_ref[...]
        x2t_ref[...] = x2t
        xcat = jnp.concatenate([x0t_ref[...], x1t, x2t],
                               axis=0).astype(jnp.bfloat16)
        v = jax.lax.dot_general(
            wru_ref[...], xcat, (((1,), (0,)), ((), ())),
            preferred_element_type=jnp.float32)
        v = jax.nn.sigmoid(v + bru_ref[...])
        rhxt = v[:fh, :] * hxt_ref[...]
        rhxt_ref[...] = rhxt
        ut_ref[...] = v[fh:, :]
        xnm_ref[:, :fh] = jnp.transpose(rhxt.astype(jnp.bfloat16), (1, 0))

    @pl.when((p == 1) & (i == last))
    def _():
        x1st = acc_ref[:fh, :] + dinv * rhxt_ref[...]
        x1st_ref[...] = x1st
        xnm_ref[:, :fh] = jnp.transpose(x1st.astype(jnp.bfloat16), (1, 0))

    @pl.when((p == 2) & (i == last))
    def _():
        rhxt = rhxt_ref[...]
        x2st = 2.0 * (acc_ref[:fh, :] + dinv * x1st_ref[...]) - rhxt
        xcat = jnp.concatenate([
            x0t_ref[:fin, :], rhxt,
            x1t_ref[:fin, :], x1st_ref[...],
            x2t_ref[:fin, :], x2st], axis=0).astype(jnp.bfloat16)
        c = jax.lax.dot_general(
            wc_ref[...], xcat, (((1,), (0,)), ((), ())),
            preferred_element_type=jnp.float32)
        c = jnp.tanh(c + bc_ref[...])
        u = ut_ref[...]
        newt = u * hxt_ref[...] + (1.0 - u) * c
        out_ref[...] = jnp.transpose(newt, (1, 0))


def kernel(inputs, hx, adj, node_index, W_ru, b_ru, W_c, b_c):
    n = adj.shape[0]
    f_in = inputs.shape[1] // n
    f_h = hx.shape[1] // n
    f0 = f_in + f_h
    bm1 = _pick_bm(n, 200)
    bm = _pick_bm(n, 400)

    x0_nm = jnp.concatenate([inputs.reshape(n, f_in), hx.reshape(n, f_h)],
                            axis=1)
    x0t = x0_nm.T
    hx_t = hx.reshape(n, f_h).T

    w_ru = (W_ru.reshape(f0, 3, 2 * f_h).transpose(1, 0, 2)
            .reshape(3 * f0, 2 * f_h).T.astype(jnp.bfloat16))
    w_c = (W_c.reshape(f0, 3, f_h).transpose(1, 0, 2)
           .reshape(3 * f0, f_h).T.astype(jnp.bfloat16))

    s1t, dinv_col, ahat = _first_pass(adj, x0_nm, bm1)
    dinv_row = dinv_col.reshape(1, n)

    new_nm = pl.pallas_call(
        functools.partial(_main_body, bm, f_in, f_h),
        grid=(3, n // bm),
        in_specs=[
            pl.BlockSpec((bm, n), lambda p, i: (i, 0)),
            pl.BlockSpec((f0, n), lambda p, i: (0, 0)),
            pl.BlockSpec((f0, n), lambda p, i: (0, 0)),
            pl.BlockSpec((1, n), lambda p, i: (0, 0)),
            pl.BlockSpec((f_h, n), lambda p, i: (0, 0)),
            pl.BlockSpec((2 * f_h, 3 * f0), lambda p, i: (0, 0)),
            pl.BlockSpec((2 * f_h, 1), lambda p, i: (0, 0)),
            pl.BlockSpec((f_h, 3 * f0), lambda p, i: (0, 0)),
            pl.BlockSpec((f_h, 1), lambda p, i: (0, 0)),
        ],
        out_specs=pl.BlockSpec((n, f_h), lambda p, i: (0, 0)),
        out_shape=jax.ShapeDtypeStruct((n, f_h), jnp.float32),
        scratch_shapes=[
            pltpu.VMEM((n, f0), jnp.bfloat16),
            pltpu.VMEM((f0, n), jnp.float32),
            pltpu.VMEM((f0, n), jnp.float32),
            pltpu.VMEM((f_h, n), jnp.float32),
            pltpu.VMEM((f_h, n), jnp.float32),
            pltpu.VMEM((f_h, n), jnp.float32),
            pltpu.VMEM((f0, n), jnp.float32),
        ],
    )(ahat, s1t, x0t, dinv_row, hx_t, w_ru, b_ru.reshape(-1, 1),
      w_c, b_c.reshape(-1, 1))

    return new_nm.reshape(1, n * f_h)
